# baseline (speedup 1.0000x reference)
import jax
import jax.numpy as jnp
from jax import lax
from jax.experimental import pallas as pl
from jax.experimental.pallas import tpu as pltpu
from jax.experimental.pallas import tpu_sc as plsc

N = 10000
E = 320000
D = 128
NC = 2
NS = 16
NW = NC * NS
PER_W = E // NW
K = 100
CHUNKS = PER_W // K
NB = 2
IDXB = CHUNKS // NB
KA = 80
PER_WA = 10080
CHUNKSA = PER_WA // KA
NBA = 3
IDXBA = CHUNKSA // NBA
EPAD = NW * PER_WA
NPAD = 10112
RPT = NPAD // NS
CW = 128


def _make_agg():
  mesh = plsc.VectorSubcoreMesh(core_axis_name="c", subcore_axis_name="s")
  scratch = [
      pltpu.VMEM_SHARED((NPAD, D), jnp.float32),
      pltpu.VMEM((IDXBA, KA), jnp.int32),
      pltpu.VMEM((IDXBA, KA), jnp.int32),
      pltpu.VMEM((KA, D), jnp.float32),
      pltpu.VMEM((KA, D), jnp.float32),
      pltpu.VMEM((KA, D), jnp.float32),
      pltpu.SemaphoreType.DMA,
      pltpu.SemaphoreType.DMA,
      pltpu.SemaphoreType.DMA,
  ]

  def body(h_hbm, src_hbm, dst_hbm, zeros_hbm, out_hbm,
           acc, sidx, didx, ra, rb, rc, sa, sb, sc_):
    c = lax.axis_index("c")
    s = lax.axis_index("s")
    wid = s * NC + c
    pltpu.sync_copy(zeros_hbm.at[pl.ds(s * RPT, RPT)],
                    acc.at[pl.ds(s * RPT, RPT)])
    plsc.subcore_barrier()

    for b in range(NBA):
      pltpu.sync_copy(src_hbm.at[wid, b], sidx)
      pltpu.sync_copy(dst_hbm.at[wid, b], didx)
      pltpu.async_copy(h_hbm.at[sidx.at[0]], ra, sa)
      pltpu.async_copy(h_hbm.at[sidx.at[1]], rb, sb)
      pltpu.async_copy(h_hbm.at[sidx.at[2]], rc, sc_)

      def step(g, carry):
        pltpu.make_async_copy(h_hbm.at[sidx.at[0]], ra, sa).wait()
        pltpu.sync_copy(ra, acc.at[didx.at[3 * g]], add=True)

        @pl.when(g < IDXBA // 3 - 1)
        def _():
          pltpu.async_copy(h_hbm.at[sidx.at[3 * g + 3]], ra, sa)

        pltpu.make_async_copy(h_hbm.at[sidx.at[0]], rb, sb).wait()
        pltpu.sync_copy(rb, acc.at[didx.at[3 * g + 1]], add=True)

        @pl.when(g < IDXBA // 3 - 1)
        def _():
          pltpu.async_copy(h_hbm.at[sidx.at[3 * g + 4]], rb, sb)

        pltpu.make_async_copy(h_hbm.at[sidx.at[0]], rc, sc_).wait()
        pltpu.sync_copy(rc, acc.at[didx.at[3 * g + 2]], add=True)

        @pl.when(g < IDXBA // 3 - 1)
        def _():
          pltpu.async_copy(h_hbm.at[sidx.at[3 * g + 5]], rc, sc_)

        return carry

      lax.fori_loop(0, IDXBA // 3, step, 0)
    plsc.subcore_barrier()
    pltpu.sync_copy(acc.at[pl.ds(s * RPT, RPT)],
                    out_hbm.at[c, pl.ds(s * RPT, RPT)])

  return pl.kernel(body,
                   out_type=jax.ShapeDtypeStruct((NC, NPAD, D), jnp.float32),
                   mesh=mesh, scratch_types=scratch)


def _make_cnt():
  mesh = plsc.VectorSubcoreMesh(core_axis_name="c", subcore_axis_name="s")
  scratch = [
      pltpu.VMEM_SHARED((NPAD, CW), jnp.float32),
      pltpu.VMEM((IDXB, K), jnp.int32),
      pltpu.VMEM((K, CW), jnp.float32),
      pltpu.SemaphoreType.DMA,
      pltpu.SemaphoreType.DMA,
  ]

  def body(dst_hbm, zeros_hbm, ones_hbm, out_hbm, accc, didx, ones_v,
           csem0, csem1):
    c = lax.axis_index("c")
    s = lax.axis_index("s")
    wid = s * NC + c
    pltpu.sync_copy(zeros_hbm.at[pl.ds(s * RPT, RPT)],
                    accc.at[pl.ds(s * RPT, RPT)])
    pltpu.sync_copy(ones_hbm, ones_v)
    plsc.subcore_barrier()

    for b in range(NB):
      pltpu.sync_copy(dst_hbm.at[wid, b], didx)

      def step(g, carry2):
        a = pltpu.async_copy(ones_v, accc.at[didx.at[2 * g]], csem0,
                             add=True)
        d = pltpu.async_copy(ones_v, accc.at[didx.at[2 * g + 1]], csem1,
                             add=True)
        a.wait()
        d.wait()
        return carry2

      lax.fori_loop(0, IDXB // 2, step, 0)
    plsc.subcore_barrier()
    pltpu.sync_copy(accc.at[pl.ds(s * RPT, RPT)],
                    out_hbm.at[c, pl.ds(s * RPT, RPT)])

  return pl.kernel(body,
                   out_type=jax.ShapeDtypeStruct((NC, NPAD, CW), jnp.float32),
                   mesh=mesh, scratch_types=scratch)


_agg = _make_agg()
_cnt = _make_cnt()


def _make_tc_layer(relu: bool):
  R = 1000

  def body(p_ref, c_ref, h_ref, wl_ref, wr_ref, b_ref, o_ref):
    agg = p_ref[0] + p_ref[1]
    cnt = c_ref[0, :, 0:1] + c_ref[1, :, 0:1]
    mean = agg / jnp.maximum(cnt, 1.0)
    acc = jnp.dot(mean, wl_ref[...], preferred_element_type=jnp.float32)
    acc += jnp.dot(h_ref[...], wr_ref[...], preferred_element_type=jnp.float32)
    acc += b_ref[...]
    if relu:
      acc = jnp.maximum(acc, 0.0)
    o_ref[...] = acc

  return pl.pallas_call(
      body,
      grid=(N // R,),
      in_specs=[
          pl.BlockSpec((NC, R, D), lambda i: (0, i, 0)),
          pl.BlockSpec((NC, R, 8), lambda i: (0, i, 0)),
          pl.BlockSpec((R, D), lambda i: (i, 0)),
          pl.BlockSpec((D, D), lambda i: (0, 0)),
          pl.BlockSpec((D, D), lambda i: (0, 0)),
          pl.BlockSpec((1, D), lambda i: (0, 0)),
      ],
      out_specs=pl.BlockSpec((R, D), lambda i: (i, 0)),
      out_shape=jax.ShapeDtypeStruct((N, D), jnp.float32),
  )


_tc_mid = _make_tc_layer(True)
_tc_last = _make_tc_layer(False)


def kernel(x, edge_index, Wl0, Wr0, b0, Wl1, Wr1, b1, Wl2, Wr2, b2):
  ei = edge_index.astype(jnp.int32)
  dst4d = ei[1].reshape(NW, NB, IDXB, K)
  pad = EPAD - E
  srcp = jnp.concatenate([ei[0], jnp.zeros((pad,), jnp.int32)])
  dstp = jnp.concatenate([ei[1], jnp.full((pad,), N, jnp.int32)])
  src4a = srcp.reshape(NW, NBA, IDXBA, KA)
  dst4a = dstp.reshape(NW, NBA, IDXBA, KA)
  zeros = jnp.zeros((NPAD, D), jnp.float32)
  ones = jnp.ones((K, CW), jnp.float32)

  cntp = _cnt(dst4d, zeros, ones)[:, :, :8]
  p0 = _agg(x, src4a, dst4a, zeros)
  h1 = _tc_mid(p0, cntp, x, Wl0, Wr0, b0.reshape(1, D))
  p1 = _agg(h1, src4a, dst4a, zeros)
  h2 = _tc_mid(p1, cntp, h1, Wl1, Wr1, b1.reshape(1, D))
  p2 = _agg(h2, src4a, dst4a, zeros)
  return _tc_last(p2, cntp, h2, Wl2, Wr2, b2.reshape(1, D))


# speedup vs baseline: 1.6441x; 1.6441x over previous
import jax
import jax.numpy as jnp
from jax import lax
from jax.experimental import pallas as pl
from jax.experimental.pallas import tpu as pltpu
from jax.experimental.pallas import tpu_sc as plsc

N = 10000
E = 320000
D = 128
NC = 2
NS = 16
NW = NC * NS
PER_W = E // NW
K = 100
CHUNKS = PER_W // K
NB = 2
IDXB = CHUNKS // NB
NPAD = 10112
RPT = NPAD // NS
CW = 128


def _make_agg():
  mesh = plsc.VectorSubcoreMesh(core_axis_name="c", subcore_axis_name="s")
  scratch = [
      pltpu.VMEM_SHARED((NPAD, D), jnp.float32),
      pltpu.VMEM((IDXB, K), jnp.int32),
      pltpu.VMEM((IDXB, K), jnp.int32),
      pltpu.VMEM((K, D), jnp.float32),
      pltpu.VMEM((K, D), jnp.float32),
      pltpu.SemaphoreType.DMA,
      pltpu.SemaphoreType.DMA,
  ]

  def body(h_hbm, src_hbm, dst_hbm, zeros_hbm, out_hbm,
           acc, sidx, didx, rows0, rows1, sem0, sem1):
    c = lax.axis_index("c")
    s = lax.axis_index("s")
    wid = s * NC + c
    pltpu.sync_copy(zeros_hbm.at[pl.ds(s * RPT, RPT)],
                    acc.at[pl.ds(s * RPT, RPT)])
    plsc.subcore_barrier()

    for b in range(NB):
      pltpu.sync_copy(src_hbm.at[wid, b], sidx)
      pltpu.sync_copy(dst_hbm.at[wid, b], didx)
      pltpu.async_copy(h_hbm.at[sidx.at[0]], rows0, sem0)

      def step(g, carry):
        pltpu.async_copy(h_hbm.at[sidx.at[2 * g + 1]], rows1, sem1)
        pltpu.make_async_copy(h_hbm.at[sidx.at[0]], rows0, sem0).wait()
        pltpu.sync_copy(rows0, acc.at[didx.at[2 * g]], add=True)

        @pl.when(g < IDXB // 2 - 1)
        def _():
          pltpu.async_copy(h_hbm.at[sidx.at[2 * g + 2]], rows0, sem0)

        pltpu.make_async_copy(h_hbm.at[sidx.at[0]], rows1, sem1).wait()
        pltpu.sync_copy(rows1, acc.at[didx.at[2 * g + 1]], add=True)
        return carry

      lax.fori_loop(0, IDXB // 2, step, 0)
    plsc.subcore_barrier()
    pltpu.sync_copy(acc.at[pl.ds(s * RPT, RPT)],
                    out_hbm.at[c, pl.ds(s * RPT, RPT)])

  return pl.kernel(body,
                   out_type=jax.ShapeDtypeStruct((NC, NPAD, D), jnp.float32),
                   mesh=mesh, scratch_types=scratch)


def _make_cnt():
  mesh = plsc.VectorSubcoreMesh(core_axis_name="c", subcore_axis_name="s")
  scratch = [
      pltpu.VMEM_SHARED((NPAD, CW), jnp.float32),
      pltpu.VMEM((IDXB, K), jnp.int32),
      pltpu.VMEM((K, CW), jnp.float32),
      pltpu.SemaphoreType.DMA,
      pltpu.SemaphoreType.DMA,
  ]

  def body(dst_hbm, zeros_hbm, ones_hbm, out_hbm, accc, didx, ones_v,
           csem0, csem1):
    c = lax.axis_index("c")
    s = lax.axis_index("s")
    wid = s * NC + c
    pltpu.sync_copy(zeros_hbm.at[pl.ds(s * RPT, RPT)],
                    accc.at[pl.ds(s * RPT, RPT)])
    pltpu.sync_copy(ones_hbm, ones_v)
    plsc.subcore_barrier()

    for b in range(NB):
      pltpu.sync_copy(dst_hbm.at[wid, b], didx)

      def step(g, carry2):
        a = pltpu.async_copy(ones_v, accc.at[didx.at[2 * g]], csem0,
                             add=True)
        d = pltpu.async_copy(ones_v, accc.at[didx.at[2 * g + 1]], csem1,
                             add=True)
        a.wait()
        d.wait()
        return carry2

      lax.fori_loop(0, IDXB // 2, step, 0)
    plsc.subcore_barrier()
    pltpu.sync_copy(accc.at[pl.ds(s * RPT, RPT)],
                    out_hbm.at[c, pl.ds(s * RPT, RPT)])

  return pl.kernel(body,
                   out_type=jax.ShapeDtypeStruct((NC, NPAD, CW), jnp.float32),
                   mesh=mesh, scratch_types=scratch)


_agg = _make_agg()
_cnt = _make_cnt()


def _make_tc_layer(relu: bool):
  R = 1000

  def body(p_ref, c_ref, h_ref, wl_ref, wr_ref, b_ref, o_ref):
    agg = p_ref[0] + p_ref[1]
    cnt = c_ref[0, :, 0:1] + c_ref[1, :, 0:1]
    mean = agg / jnp.maximum(cnt, 1.0)
    acc = jnp.dot(mean, wl_ref[...], preferred_element_type=jnp.float32)
    acc += jnp.dot(h_ref[...], wr_ref[...], preferred_element_type=jnp.float32)
    acc += b_ref[...]
    if relu:
      acc = jnp.maximum(acc, 0.0)
    o_ref[...] = acc

  return pl.pallas_call(
      body,
      grid=(N // R,),
      in_specs=[
          pl.BlockSpec((NC, R, D), lambda i: (0, i, 0)),
          pl.BlockSpec((NC, R, 8), lambda i: (0, i, 0)),
          pl.BlockSpec((R, D), lambda i: (i, 0)),
          pl.BlockSpec((D, D), lambda i: (0, 0)),
          pl.BlockSpec((D, D), lambda i: (0, 0)),
          pl.BlockSpec((1, D), lambda i: (0, 0)),
      ],
      out_specs=pl.BlockSpec((R, D), lambda i: (i, 0)),
      out_shape=jax.ShapeDtypeStruct((N, D), jnp.float32),
  )


_tc_mid = _make_tc_layer(True)
_tc_last = _make_tc_layer(False)


def kernel(x, edge_index, Wl0, Wr0, b0, Wl1, Wr1, b1, Wl2, Wr2, b2):
  ei = edge_index.astype(jnp.int32)
  src4d = ei[0].reshape(NW, NB, IDXB, K)
  dst4d = ei[1].reshape(NW, NB, IDXB, K)
  zeros = jnp.zeros((NPAD, D), jnp.float32)
  ones = jnp.ones((K, CW), jnp.float32)

  cntp = _cnt(dst4d, zeros, ones)[:, :, :8]
  p0 = _agg(x, src4d, dst4d, zeros)
  h1 = _tc_mid(p0, cntp, x, Wl0, Wr0, b0.reshape(1, D))
  p1 = _agg(h1, src4d, dst4d, zeros)
  h2 = _tc_mid(p1, cntp, h1, Wl1, Wr1, b1.reshape(1, D))
  p2 = _agg(h2, src4d, dst4d, zeros)
  return _tc_last(p2, cntp, h2, Wl2, Wr2, b2.reshape(1, D))


# speedup vs baseline: 1.6721x; 1.0171x over previous
import jax
import jax.numpy as jnp
from jax import lax
from jax.experimental import pallas as pl
from jax.experimental.pallas import tpu as pltpu
from jax.experimental.pallas import tpu_sc as plsc

N = 10000
E = 320000
D = 128
NC = 2
NS = 16
NW = NC * NS
PER_W = E // NW
K = 100
CHUNKS = PER_W // K
NB = 2
IDXB = CHUNKS // NB
NPAD = 10112
RPT = NPAD // NS
CW = 128


def _make_agg():
  mesh = plsc.VectorSubcoreMesh(core_axis_name="c", subcore_axis_name="s")
  scratch = [
      pltpu.VMEM_SHARED((NPAD, D), jnp.float32),
      pltpu.VMEM((IDXB, K), jnp.int32),
      pltpu.VMEM((IDXB, K), jnp.int32),
      pltpu.VMEM((K, D), jnp.float32),
      pltpu.VMEM((K, D), jnp.float32),
      pltpu.SemaphoreType.DMA,
      pltpu.SemaphoreType.DMA,
  ]

  def body(h_hbm, src_hbm, dst_hbm, zeros_hbm, out_hbm,
           acc, sidx, didx, rows0, rows1, sem0, sem1):
    c = lax.axis_index("c")
    s = lax.axis_index("s")
    wid = s * NC + c
    pltpu.sync_copy(zeros_hbm.at[pl.ds(s * RPT, RPT)],
                    acc.at[pl.ds(s * RPT, RPT)])
    plsc.subcore_barrier()

    for b in range(NB):
      pltpu.sync_copy(src_hbm.at[wid, b], sidx)
      pltpu.sync_copy(dst_hbm.at[wid, b], didx)
      pltpu.async_copy(h_hbm.at[sidx.at[0]], rows0, sem0)

      def step(g, carry):
        pltpu.async_copy(h_hbm.at[sidx.at[2 * g + 1]], rows1, sem1)
        pltpu.make_async_copy(h_hbm.at[sidx.at[0]], rows0, sem0).wait()
        pltpu.sync_copy(rows0, acc.at[didx.at[2 * g]], add=True)

        @pl.when(g < IDXB // 2 - 1)
        def _():
          pltpu.async_copy(h_hbm.at[sidx.at[2 * g + 2]], rows0, sem0)

        pltpu.make_async_copy(h_hbm.at[sidx.at[0]], rows1, sem1).wait()
        pltpu.sync_copy(rows1, acc.at[didx.at[2 * g + 1]], add=True)
        return carry

      lax.fori_loop(0, IDXB // 2, step, 0)
    plsc.subcore_barrier()
    pltpu.sync_copy(acc.at[pl.ds(s * RPT, RPT)],
                    out_hbm.at[c, pl.ds(s * RPT, RPT)])

  return pl.kernel(body,
                   out_type=jax.ShapeDtypeStruct((NC, NPAD, D), jnp.float32),
                   mesh=mesh, scratch_types=scratch)


def _make_cnt():
  mesh = plsc.VectorSubcoreMesh(core_axis_name="c", subcore_axis_name="s")
  scratch = [
      pltpu.VMEM_SHARED((NPAD, CW), jnp.float32),
      pltpu.VMEM((IDXB, K), jnp.int32),
      pltpu.VMEM((K, CW), jnp.float32),
      pltpu.SemaphoreType.DMA,
      pltpu.SemaphoreType.DMA,
  ]

  def body(dst_hbm, zeros_hbm, ones_hbm, out_hbm, accc, didx, ones_v,
           csem0, csem1):
    c = lax.axis_index("c")
    s = lax.axis_index("s")
    wid = s * NC + c
    pltpu.sync_copy(zeros_hbm.at[pl.ds(s * RPT, RPT)],
                    accc.at[pl.ds(s * RPT, RPT)])
    pltpu.sync_copy(ones_hbm, ones_v)
    plsc.subcore_barrier()

    for b in range(NB):
      pltpu.sync_copy(dst_hbm.at[wid, b], didx)

      def step(g, carry2):
        a = pltpu.async_copy(ones_v, accc.at[didx.at[2 * g]], csem0,
                             add=True)
        d = pltpu.async_copy(ones_v, accc.at[didx.at[2 * g + 1]], csem1,
                             add=True)
        a.wait()
        d.wait()
        return carry2

      lax.fori_loop(0, IDXB // 2, step, 0)
    plsc.subcore_barrier()
    pltpu.sync_copy(accc.at[pl.ds(s * RPT, RPT)],
                    out_hbm.at[c, pl.ds(s * RPT, RPT)])

  return pl.kernel(body,
                   out_type=jax.ShapeDtypeStruct((NC, NPAD, CW), jnp.float32),
                   mesh=mesh, scratch_types=scratch)


_agg = _make_agg()
_cnt = _make_cnt()


def _make_tc_layer(relu: bool):
  R = 2000

  def body(p_ref, c_ref, h_ref, wl_ref, wr_ref, b_ref, o_ref):
    agg = p_ref[0] + p_ref[1]
    cnt = c_ref[0, :, 0:1] + c_ref[1, :, 0:1]
    mean = agg / jnp.maximum(cnt, 1.0)
    acc = jnp.dot(mean, wl_ref[...], preferred_element_type=jnp.float32)
    acc += jnp.dot(h_ref[...], wr_ref[...], preferred_element_type=jnp.float32)
    acc += b_ref[...]
    if relu:
      acc = jnp.maximum(acc, 0.0)
    o_ref[...] = acc

  return pl.pallas_call(
      body,
      grid=(N // R,),
      in_specs=[
          pl.BlockSpec((NC, R, D), lambda i: (0, i, 0)),
          pl.BlockSpec((NC, R, 8), lambda i: (0, i, 0)),
          pl.BlockSpec((R, D), lambda i: (i, 0)),
          pl.BlockSpec((D, D), lambda i: (0, 0)),
          pl.BlockSpec((D, D), lambda i: (0, 0)),
          pl.BlockSpec((1, D), lambda i: (0, 0)),
      ],
      out_specs=pl.BlockSpec((R, D), lambda i: (i, 0)),
      out_shape=jax.ShapeDtypeStruct((N, D), jnp.float32),
  )


_tc_mid = _make_tc_layer(True)
_tc_last = _make_tc_layer(False)


def kernel(x, edge_index, Wl0, Wr0, b0, Wl1, Wr1, b1, Wl2, Wr2, b2):
  ei = edge_index.astype(jnp.int32)
  src4d = ei[0].reshape(NW, NB, IDXB, K)
  dst4d = ei[1].reshape(NW, NB, IDXB, K)
  zeros = jnp.zeros((NPAD, D), jnp.float32)
  ones = jnp.ones((K, CW), jnp.float32)

  cntp = _cnt(dst4d, zeros, ones)[:, :, :8]
  p0 = _agg(x, src4d, dst4d, zeros)
  h1 = _tc_mid(p0, cntp, x, Wl0, Wr0, b0.reshape(1, D))
  p1 = _agg(h1, src4d, dst4d, zeros)
  h2 = _tc_mid(p1, cntp, h1, Wl1, Wr1, b1.reshape(1, D))
  p2 = _agg(h2, src4d, dst4d, zeros)
  return _tc_last(p2, cntp, h2, Wl2, Wr2, b2.reshape(1, D))
